# baseline (speedup 1.0000x reference)
import functools

import jax
import jax.numpy as jnp
from jax.experimental import pallas as pl
from jax.experimental.pallas import tpu as pltpu

B, L, D = 4, 128, 768
SPAN_PRUNING = 0.3
ROWS = B * L * L
BLK = 8


def _proj_body(t_ref, w_ref, b_ref, s_ref, e_ref):
    x = t_ref[...]
    x2 = x.reshape(BLK * L, D)
    r = jnp.dot(x2, w_ref[...], preferred_element_type=jnp.float32)
    r = r + b_ref[...]
    s_ref[...] = r[:, 0].reshape(BLK, L)
    e_ref[...] = r[:, 1].reshape(BLK, L)


def _bce_elem(logits, targets):
    return (jnp.maximum(logits, 0.0) - logits * targets
            + jnp.log1p(jnp.exp(-jnp.abs(logits))))


def _kth_largest_bits(p_bits, k, count_fn, n_iter=31):
    lo0 = jnp.zeros_like(k)
    hi0 = jnp.full_like(k, 0x3F800000)

    def body(_, lohi):
        lo, hi = lohi
        mid = lo + (hi - lo + 1) // 2
        ge = count_fn(mid) >= k
        return jnp.where(ge, mid, lo), jnp.where(ge, hi, mid - 1)

    lo, _ = jax.lax.fori_loop(0, n_iter, body, (lo0, hi0))
    return lo


def _head_body(ls_ref, le_ref, labs_ref, labe_ref, ia_ref, labias_ref,
               labiae_ref, am_ref, wia_ref, bia_ref,
               loss_s_ref, loss_e_ref, loss_ias_ref, loss_iae_ref,
               ms_ref, me_ref, mias_ref, miae_ref):
    am = am_ref[...]
    msum = jnp.sum(am, axis=1, keepdims=True)
    ml = msum - 3.0
    ln = (ml * SPAN_PRUNING).astype(jnp.int32)
    ln = jnp.maximum(ln, 10)
    maxl = (ml * ml).astype(jnp.int32)
    k = jnp.minimum(ln, maxl)

    def table_head(l_ref, lab_ref, loss_ref, m_ref):
        logits = l_ref[...]
        lab = lab_ref[...]
        w = (lab >= 0).astype(jnp.float32)
        elem = _bce_elem(logits, lab.astype(jnp.float32))
        loss_ref[...] = jnp.sum(w * elem).reshape(1, 1) / float(B * L * L)
        p = jax.nn.sigmoid(logits) * w
        p3 = p.reshape(B, L, L)
        pb = jax.lax.bitcast_convert_type(p3, jnp.int32)

        def count(mid):
            ge = jnp.where(pb >= mid[:, :, None], 1, 0)
            return jnp.sum(jnp.sum(ge, axis=2), axis=1, keepdims=True)

        thr_bits = _kth_largest_bits(pb, k, count)
        thr = jax.lax.bitcast_convert_type(thr_bits, jnp.float32)[:, :, None]
        strict = (thr[0:1, :, :] == 0.0)
        gt = jnp.where(p3 > thr, 1.0, 0.0)
        ge = jnp.where(p3 >= thr, 1.0, 0.0)
        m_ref[...] = jnp.where(strict, gt, ge).reshape(B * L, L)

    table_head(ls_ref, labs_ref, loss_s_ref, ms_ref)
    table_head(le_ref, labe_ref, loss_e_ref, me_ref)

    x = ia_ref[...]
    wia = wia_ref[...]
    bia = bia_ref[...]

    x16 = x.astype(jnp.bfloat16).astype(jnp.float32)

    def ia_head(col, lab_ref, loss_ref, m_ref):
        wvec = wia[0, col * D:(col + 1) * D].reshape(1, 1, D)
        wvec = wvec.astype(jnp.bfloat16).astype(jnp.float32)
        logits = jnp.sum(x16 * wvec, axis=2) + bia[0, col]
        lab = lab_ref[...]
        w = (lab >= 0).astype(jnp.float32)
        elem = _bce_elem(logits, lab.astype(jnp.float32))
        loss_ref[...] = jnp.sum(w * elem).reshape(1, 1) / float(B * L)
        p = jax.nn.sigmoid(logits) * w
        pb = jax.lax.bitcast_convert_type(p, jnp.int32)

        def count(mid):
            return jnp.sum(jnp.where(pb >= mid, 1, 0), axis=1, keepdims=True)

        thr_bits = _kth_largest_bits(pb, k, count)
        thr = jax.lax.bitcast_convert_type(thr_bits, jnp.float32)
        p2 = p[None, :, :]
        thr3 = thr[:, :, None]
        strict = (thr3[0:1, :, :] == 0.0)
        gt = jnp.where(p2 > thr3, 1.0, 0.0)
        ge = jnp.where(p2 >= thr3, 1.0, 0.0)
        m_ref[...] = jnp.where(strict, gt, ge)

    ia_head(0, labias_ref, loss_ias_ref, mias_ref)
    ia_head(1, labiae_ref, loss_iae_ref, miae_ref)


@functools.partial(jax.jit, static_argnames=())
def _run(table, attention_mask, table_labels_S, table_labels_E,
         table_labels_iaS, table_labels_iaE, ia_seq,
         W_S, b_S, W_E, b_E, W_iaS, b_iaS, W_iaE, b_iaE):
    t3 = table.reshape(B * L, L, D)
    wc = jnp.concatenate([W_S, W_E], axis=1)
    bc = jnp.concatenate([b_S, b_E]).reshape(1, 2)

    nblk = (B * L) // BLK
    logits_S, logits_E = pl.pallas_call(
        _proj_body,
        grid=(nblk,),
        in_specs=[
            pl.BlockSpec((BLK, L, D), lambda g: (g, 0, 0)),
            pl.BlockSpec((D, 2), lambda g: (0, 0)),
            pl.BlockSpec((1, 2), lambda g: (0, 0)),
        ],
        out_specs=[
            pl.BlockSpec((BLK, L), lambda g: (g, 0)),
            pl.BlockSpec((BLK, L), lambda g: (g, 0)),
        ],
        out_shape=[
            jax.ShapeDtypeStruct((B * L, L), jnp.float32),
            jax.ShapeDtypeStruct((B * L, L), jnp.float32),
        ],
        compiler_params=pltpu.CompilerParams(
            dimension_semantics=("parallel",),
        ),
    )(t3, wc, bc)

    wia = jnp.concatenate([W_iaS[:, 0], W_iaE[:, 0]]).reshape(1, 2 * D)
    bia = jnp.concatenate([b_iaS, b_iaE]).reshape(1, 2)

    outs = pl.pallas_call(
        _head_body,
        out_shape=[
            jax.ShapeDtypeStruct((1, 1), jnp.float32),
            jax.ShapeDtypeStruct((1, 1), jnp.float32),
            jax.ShapeDtypeStruct((1, 1), jnp.float32),
            jax.ShapeDtypeStruct((1, 1), jnp.float32),
            jax.ShapeDtypeStruct((B * L, L), jnp.float32),
            jax.ShapeDtypeStruct((B * L, L), jnp.float32),
            jax.ShapeDtypeStruct((B, B, L), jnp.float32),
            jax.ShapeDtypeStruct((B, B, L), jnp.float32),
        ],
    )(logits_S, logits_E,
      table_labels_S.reshape(B * L, L), table_labels_E.reshape(B * L, L),
      ia_seq, table_labels_iaS, table_labels_iaE, attention_mask, wia, bia)

    loss_S, loss_E, loss_iaS, loss_iaE, mS, mE, miaS, miaE = outs
    return (loss_S[0, 0], loss_E[0, 0], loss_iaS[0, 0], loss_iaE[0, 0],
            mS.reshape(B, L, L).astype(jnp.bool_),
            mE.reshape(B, L, L).astype(jnp.bool_),
            miaS.astype(jnp.bool_), miaE.astype(jnp.bool_))


def kernel(table, attention_mask, table_labels_S, table_labels_E,
           table_labels_iaS, table_labels_iaE, ia_seq,
           W_S, b_S, W_E, b_E, W_iaS, b_iaS, W_iaE, b_iaE):
    return _run(table, attention_mask, table_labels_S, table_labels_E,
                table_labels_iaS, table_labels_iaE, ia_seq,
                W_S, b_S, W_E, b_E, W_iaS, b_iaS, W_iaE, b_iaE)


# speedup vs baseline: 1.1717x; 1.1717x over previous
import functools

import jax
import jax.numpy as jnp
from jax.experimental import pallas as pl
from jax.experimental.pallas import tpu as pltpu

B, L, D = 4, 128, 768
SPAN_PRUNING = 0.3
ROWS = B * L * L
BLK = 16


def _proj_body(t_ref, w_ref, b_ref, s_ref, e_ref):
    x = t_ref[...]
    x2 = x.reshape(BLK * L, D)
    r = jnp.dot(x2, w_ref[...], preferred_element_type=jnp.float32)
    r = r + b_ref[...]
    s_ref[...] = r[:, 0].reshape(BLK, L)
    e_ref[...] = r[:, 1].reshape(BLK, L)


def _bce_elem(logits, targets):
    return (jnp.maximum(logits, 0.0) - logits * targets
            + jnp.log1p(jnp.exp(-jnp.abs(logits))))


def _kth_largest_bits(p_bits, k, count_fn, n_iter=31):
    lo0 = jnp.zeros_like(k)
    hi0 = jnp.full_like(k, 0x3F800000)

    def body(_, lohi):
        lo, hi = lohi
        mid = lo + (hi - lo + 1) // 2
        ge = count_fn(mid) >= k
        return jnp.where(ge, mid, lo), jnp.where(ge, hi, mid - 1)

    lo, _ = jax.lax.fori_loop(0, n_iter, body, (lo0, hi0))
    return lo


def _head_body(ls_ref, le_ref, labs_ref, labe_ref, ia_ref, labias_ref,
               labiae_ref, am_ref, wia_ref, bia_ref,
               loss_s_ref, loss_e_ref, loss_ias_ref, loss_iae_ref,
               ms_ref, me_ref, mias_ref, miae_ref):
    am = am_ref[...]
    msum = jnp.sum(am, axis=1, keepdims=True)
    ml = msum - 3.0
    ln = (ml * SPAN_PRUNING).astype(jnp.int32)
    ln = jnp.maximum(ln, 10)
    maxl = (ml * ml).astype(jnp.int32)
    k = jnp.minimum(ln, maxl)

    def table_head(l_ref, lab_ref, loss_ref, m_ref):
        logits = l_ref[...]
        lab = lab_ref[...]
        w = (lab >= 0).astype(jnp.float32)
        elem = _bce_elem(logits, lab.astype(jnp.float32))
        loss_ref[...] = jnp.sum(w * elem).reshape(1, 1) / float(B * L * L)
        p = jax.nn.sigmoid(logits) * w
        p3 = p.reshape(B, L, L)
        pb = jax.lax.bitcast_convert_type(p3, jnp.int32)

        def count(mid):
            ge = jnp.where(pb >= mid[:, :, None], 1, 0)
            return jnp.sum(jnp.sum(ge, axis=2), axis=1, keepdims=True)

        thr_bits = _kth_largest_bits(pb, k, count)
        thr = jax.lax.bitcast_convert_type(thr_bits, jnp.float32)[:, :, None]
        strict = (thr[0:1, :, :] == 0.0)
        gt = jnp.where(p3 > thr, 1.0, 0.0)
        ge = jnp.where(p3 >= thr, 1.0, 0.0)
        m_ref[...] = jnp.where(strict, gt, ge).reshape(B * L, L)

    table_head(ls_ref, labs_ref, loss_s_ref, ms_ref)
    table_head(le_ref, labe_ref, loss_e_ref, me_ref)

    x = ia_ref[...]
    wia = wia_ref[...]
    bia = bia_ref[...]

    x16 = x.astype(jnp.bfloat16).astype(jnp.float32)

    def ia_head(col, lab_ref, loss_ref, m_ref):
        wvec = wia[0, col * D:(col + 1) * D].reshape(1, 1, D)
        wvec = wvec.astype(jnp.bfloat16).astype(jnp.float32)
        logits = jnp.sum(x16 * wvec, axis=2) + bia[0, col]
        lab = lab_ref[...]
        w = (lab >= 0).astype(jnp.float32)
        elem = _bce_elem(logits, lab.astype(jnp.float32))
        loss_ref[...] = jnp.sum(w * elem).reshape(1, 1) / float(B * L)
        p = jax.nn.sigmoid(logits) * w
        pb = jax.lax.bitcast_convert_type(p, jnp.int32)

        def count(mid):
            return jnp.sum(jnp.where(pb >= mid, 1, 0), axis=1, keepdims=True)

        thr_bits = _kth_largest_bits(pb, k, count)
        thr = jax.lax.bitcast_convert_type(thr_bits, jnp.float32)
        p2 = p[None, :, :]
        thr3 = thr[:, :, None]
        strict = (thr3[0:1, :, :] == 0.0)
        gt = jnp.where(p2 > thr3, 1.0, 0.0)
        ge = jnp.where(p2 >= thr3, 1.0, 0.0)
        m_ref[...] = jnp.where(strict, gt, ge)

    ia_head(0, labias_ref, loss_ias_ref, mias_ref)
    ia_head(1, labiae_ref, loss_iae_ref, miae_ref)


@functools.partial(jax.jit, static_argnames=())
def _run(table, attention_mask, table_labels_S, table_labels_E,
         table_labels_iaS, table_labels_iaE, ia_seq,
         W_S, b_S, W_E, b_E, W_iaS, b_iaS, W_iaE, b_iaE):
    t3 = table.reshape(B * L, L, D)
    wc = jnp.concatenate([W_S, W_E], axis=1)
    bc = jnp.concatenate([b_S, b_E]).reshape(1, 2)

    nblk = (B * L) // BLK
    logits_S, logits_E = pl.pallas_call(
        _proj_body,
        grid=(nblk,),
        in_specs=[
            pl.BlockSpec((BLK, L, D), lambda g: (g, 0, 0)),
            pl.BlockSpec((D, 2), lambda g: (0, 0)),
            pl.BlockSpec((1, 2), lambda g: (0, 0)),
        ],
        out_specs=[
            pl.BlockSpec((BLK, L), lambda g: (g, 0)),
            pl.BlockSpec((BLK, L), lambda g: (g, 0)),
        ],
        out_shape=[
            jax.ShapeDtypeStruct((B * L, L), jnp.float32),
            jax.ShapeDtypeStruct((B * L, L), jnp.float32),
        ],
        compiler_params=pltpu.CompilerParams(
            dimension_semantics=("parallel",),
        ),
    )(t3, wc, bc)

    wia = jnp.concatenate([W_iaS[:, 0], W_iaE[:, 0]]).reshape(1, 2 * D)
    bia = jnp.concatenate([b_iaS, b_iaE]).reshape(1, 2)

    outs = pl.pallas_call(
        _head_body,
        out_shape=[
            jax.ShapeDtypeStruct((1, 1), jnp.float32),
            jax.ShapeDtypeStruct((1, 1), jnp.float32),
            jax.ShapeDtypeStruct((1, 1), jnp.float32),
            jax.ShapeDtypeStruct((1, 1), jnp.float32),
            jax.ShapeDtypeStruct((B * L, L), jnp.float32),
            jax.ShapeDtypeStruct((B * L, L), jnp.float32),
            jax.ShapeDtypeStruct((B, B, L), jnp.float32),
            jax.ShapeDtypeStruct((B, B, L), jnp.float32),
        ],
    )(logits_S, logits_E,
      table_labels_S.reshape(B * L, L), table_labels_E.reshape(B * L, L),
      ia_seq, table_labels_iaS, table_labels_iaE, attention_mask, wia, bia)

    loss_S, loss_E, loss_iaS, loss_iaE, mS, mE, miaS, miaE = outs
    return (loss_S[0, 0], loss_E[0, 0], loss_iaS[0, 0], loss_iaE[0, 0],
            mS.reshape(B, L, L).astype(jnp.bool_),
            mE.reshape(B, L, L).astype(jnp.bool_),
            miaS.astype(jnp.bool_), miaE.astype(jnp.bool_))


def kernel(table, attention_mask, table_labels_S, table_labels_E,
           table_labels_iaS, table_labels_iaE, ia_seq,
           W_S, b_S, W_E, b_E, W_iaS, b_iaS, W_iaE, b_iaE):
    return _run(table, attention_mask, table_labels_S, table_labels_E,
                table_labels_iaS, table_labels_iaE, ia_seq,
                W_S, b_S, W_E, b_E, W_iaS, b_iaS, W_iaE, b_iaE)


# speedup vs baseline: 1.2349x; 1.0540x over previous
import functools

import jax
import jax.numpy as jnp
from jax.experimental import pallas as pl
from jax.experimental.pallas import tpu as pltpu

B, L, D = 4, 128, 768
SPAN_PRUNING = 0.3
ROWS = B * L * L
BLK = 32


def _proj_body(t_ref, w_ref, b_ref, s_ref, e_ref):
    x = t_ref[...]
    x2 = x.reshape(BLK * L, D)
    r = jnp.dot(x2, w_ref[...], preferred_element_type=jnp.float32)
    r = r + b_ref[...]
    s_ref[...] = r[:, 0].reshape(BLK, L)
    e_ref[...] = r[:, 1].reshape(BLK, L)


def _bce_elem(logits, targets):
    return (jnp.maximum(logits, 0.0) - logits * targets
            + jnp.log1p(jnp.exp(-jnp.abs(logits))))


def _kth_largest_bits(p_bits, k, count_fn, n_iter=31):
    lo0 = jnp.zeros_like(k)
    hi0 = jnp.full_like(k, 0x3F800000)

    def body(_, lohi):
        lo, hi = lohi
        mid = lo + (hi - lo + 1) // 2
        ge = count_fn(mid) >= k
        return jnp.where(ge, mid, lo), jnp.where(ge, hi, mid - 1)

    lo, _ = jax.lax.fori_loop(0, n_iter, body, (lo0, hi0))
    return lo


def _head_body(ls_ref, le_ref, labs_ref, labe_ref, ia_ref, labias_ref,
               labiae_ref, am_ref, wia_ref, bia_ref,
               loss_s_ref, loss_e_ref, loss_ias_ref, loss_iae_ref,
               ms_ref, me_ref, mias_ref, miae_ref):
    am = am_ref[...]
    msum = jnp.sum(am, axis=1, keepdims=True)
    ml = msum - 3.0
    ln = (ml * SPAN_PRUNING).astype(jnp.int32)
    ln = jnp.maximum(ln, 10)
    maxl = (ml * ml).astype(jnp.int32)
    k = jnp.minimum(ln, maxl)

    def table_head(l_ref, lab_ref, loss_ref, m_ref):
        logits = l_ref[...]
        lab = lab_ref[...]
        w = (lab >= 0).astype(jnp.float32)
        elem = _bce_elem(logits, lab.astype(jnp.float32))
        loss_ref[...] = jnp.sum(w * elem).reshape(1, 1) / float(B * L * L)
        p = jax.nn.sigmoid(logits) * w
        p3 = p.reshape(B, L, L)
        pb = jax.lax.bitcast_convert_type(p3, jnp.int32)

        def count(mid):
            ge = jnp.where(pb >= mid[:, :, None], 1, 0)
            return jnp.sum(jnp.sum(ge, axis=2), axis=1, keepdims=True)

        thr_bits = _kth_largest_bits(pb, k, count)
        thr = jax.lax.bitcast_convert_type(thr_bits, jnp.float32)[:, :, None]
        strict = (thr[0:1, :, :] == 0.0)
        gt = jnp.where(p3 > thr, 1.0, 0.0)
        ge = jnp.where(p3 >= thr, 1.0, 0.0)
        m_ref[...] = jnp.where(strict, gt, ge).reshape(B * L, L)

    table_head(ls_ref, labs_ref, loss_s_ref, ms_ref)
    table_head(le_ref, labe_ref, loss_e_ref, me_ref)

    x = ia_ref[...]
    wia = wia_ref[...]
    bia = bia_ref[...]

    x16 = x.astype(jnp.bfloat16).astype(jnp.float32)

    def ia_head(col, lab_ref, loss_ref, m_ref):
        wvec = wia[0, col * D:(col + 1) * D].reshape(1, 1, D)
        wvec = wvec.astype(jnp.bfloat16).astype(jnp.float32)
        logits = jnp.sum(x16 * wvec, axis=2) + bia[0, col]
        lab = lab_ref[...]
        w = (lab >= 0).astype(jnp.float32)
        elem = _bce_elem(logits, lab.astype(jnp.float32))
        loss_ref[...] = jnp.sum(w * elem).reshape(1, 1) / float(B * L)
        p = jax.nn.sigmoid(logits) * w
        pb = jax.lax.bitcast_convert_type(p, jnp.int32)

        def count(mid):
            return jnp.sum(jnp.where(pb >= mid, 1, 0), axis=1, keepdims=True)

        thr_bits = _kth_largest_bits(pb, k, count)
        thr = jax.lax.bitcast_convert_type(thr_bits, jnp.float32)
        p2 = p[None, :, :]
        thr3 = thr[:, :, None]
        strict = (thr3[0:1, :, :] == 0.0)
        gt = jnp.where(p2 > thr3, 1.0, 0.0)
        ge = jnp.where(p2 >= thr3, 1.0, 0.0)
        m_ref[...] = jnp.where(strict, gt, ge)

    ia_head(0, labias_ref, loss_ias_ref, mias_ref)
    ia_head(1, labiae_ref, loss_iae_ref, miae_ref)


@functools.partial(jax.jit, static_argnames=())
def _run(table, attention_mask, table_labels_S, table_labels_E,
         table_labels_iaS, table_labels_iaE, ia_seq,
         W_S, b_S, W_E, b_E, W_iaS, b_iaS, W_iaE, b_iaE):
    t3 = table.reshape(B * L, L, D)
    wc = jnp.concatenate([W_S, W_E], axis=1)
    bc = jnp.concatenate([b_S, b_E]).reshape(1, 2)

    nblk = (B * L) // BLK
    logits_S, logits_E = pl.pallas_call(
        _proj_body,
        grid=(nblk,),
        in_specs=[
            pl.BlockSpec((BLK, L, D), lambda g: (g, 0, 0)),
            pl.BlockSpec((D, 2), lambda g: (0, 0)),
            pl.BlockSpec((1, 2), lambda g: (0, 0)),
        ],
        out_specs=[
            pl.BlockSpec((BLK, L), lambda g: (g, 0)),
            pl.BlockSpec((BLK, L), lambda g: (g, 0)),
        ],
        out_shape=[
            jax.ShapeDtypeStruct((B * L, L), jnp.float32),
            jax.ShapeDtypeStruct((B * L, L), jnp.float32),
        ],
        compiler_params=pltpu.CompilerParams(
            dimension_semantics=("parallel",),
        ),
    )(t3, wc, bc)

    wia = jnp.concatenate([W_iaS[:, 0], W_iaE[:, 0]]).reshape(1, 2 * D)
    bia = jnp.concatenate([b_iaS, b_iaE]).reshape(1, 2)

    outs = pl.pallas_call(
        _head_body,
        out_shape=[
            jax.ShapeDtypeStruct((1, 1), jnp.float32),
            jax.ShapeDtypeStruct((1, 1), jnp.float32),
            jax.ShapeDtypeStruct((1, 1), jnp.float32),
            jax.ShapeDtypeStruct((1, 1), jnp.float32),
            jax.ShapeDtypeStruct((B * L, L), jnp.float32),
            jax.ShapeDtypeStruct((B * L, L), jnp.float32),
            jax.ShapeDtypeStruct((B, B, L), jnp.float32),
            jax.ShapeDtypeStruct((B, B, L), jnp.float32),
        ],
    )(logits_S, logits_E,
      table_labels_S.reshape(B * L, L), table_labels_E.reshape(B * L, L),
      ia_seq, table_labels_iaS, table_labels_iaE, attention_mask, wia, bia)

    loss_S, loss_E, loss_iaS, loss_iaE, mS, mE, miaS, miaE = outs
    return (loss_S[0, 0], loss_E[0, 0], loss_iaS[0, 0], loss_iaE[0, 0],
            mS.reshape(B, L, L).astype(jnp.bool_),
            mE.reshape(B, L, L).astype(jnp.bool_),
            miaS.astype(jnp.bool_), miaE.astype(jnp.bool_))


def kernel(table, attention_mask, table_labels_S, table_labels_E,
           table_labels_iaS, table_labels_iaE, ia_seq,
           W_S, b_S, W_E, b_E, W_iaS, b_iaS, W_iaE, b_iaE):
    return _run(table, attention_mask, table_labels_S, table_labels_E,
                table_labels_iaS, table_labels_iaE, ia_seq,
                W_S, b_S, W_E, b_E, W_iaS, b_iaS, W_iaE, b_iaE)
